# Initial kernel scaffold; baseline (speedup 1.0000x reference)
import functools

import jax
import jax.numpy as jnp
from jax import lax
from jax.experimental import pallas as pl
from jax.experimental.pallas import tpu as pltpu
from jax.experimental.pallas import tpu_sc as plsc

N = 10000
E = 320000
NC = 2
NS = 16
NW = NC * NS
CHUNK = 128
CPT = 79
EPAD = NW * CPT * CHUNK
NCHUNKS = EPAD // CHUNK
NPAD = 10112
SLAB = NPAD // NS

_mesh = plsc.VectorSubcoreMesh(core_axis_name="c", subcore_axis_name="s")


def _wid_base(cpt):
    c = lax.axis_index("c")
    s = lax.axis_index("s")
    return (c * NS + s) * cpt, c, s


def _zero_vmem_2d(buf, rows, cols):
    z = jnp.zeros((16,), jnp.float32)

    def body(r, _):
        for cc in range(cols // 16):
            buf[r, pl.ds(cc * 16, 16)] = z
        return 0

    lax.fori_loop(0, rows, body, 0)


@functools.partial(
    pl.kernel,
    out_type=jax.ShapeDtypeStruct((NC, NPAD), jnp.float32),
    mesh=_mesh,
    scratch_types=[
        pltpu.VMEM((CPT, CHUNK), jnp.int32),
        pltpu.VMEM((CHUNK,), jnp.float32),
        pltpu.VMEM((SLAB,), jnp.float32),
        pltpu.VMEM_SHARED((NPAD,), jnp.float32),
    ],
)
def _sc_deg(dst_hbm, deg_hbm, dstv, ones_v, zslab, dacc):
    base, c, s = _wid_base(CPT)
    pltpu.sync_copy(dst_hbm.at[pl.ds(base, CPT)], dstv)
    one = jnp.ones((16,), jnp.float32)
    z = jnp.zeros((16,), jnp.float32)
    for i in range(CHUNK // 16):
        ones_v[pl.ds(i * 16, 16)] = one

    def zbody(i, _):
        zslab[pl.ds(i * 16, 16)] = z
        return 0

    lax.fori_loop(0, SLAB // 16, zbody, 0)
    pltpu.sync_copy(zslab, dacc.at[pl.ds(s * SLAB, SLAB)])
    plsc.subcore_barrier()

    def body(i, _):
        pltpu.sync_copy(ones_v, dacc.at[dstv.at[i]], add=True)
        return 0

    lax.fori_loop(0, CPT, body, 0)
    plsc.subcore_barrier()
    pltpu.sync_copy(dacc.at[pl.ds(s * SLAB, SLAB)], deg_hbm.at[c, pl.ds(s * SLAB, SLAB)])


def _make_sc_agg(D):
    @functools.partial(
        pl.kernel,
        out_type=jax.ShapeDtypeStruct((NC, NPAD, D), jnp.float32),
        mesh=_mesh,
        scratch_types=[
            pltpu.VMEM((CPT, CHUNK), jnp.int32),
            pltpu.VMEM((CPT, CHUNK), jnp.int32),
            pltpu.VMEM((CHUNK, D), jnp.float32),
            pltpu.VMEM_SHARED((NPAD, D), jnp.float32),
            pltpu.SemaphoreType.DMA,
        ],
    )
    def _sc_agg(h_hbm, src_hbm, dst_hbm, acc_hbm, srcv, dstv, buf, accs, sem):
        base, c, s = _wid_base(CPT)
        pltpu.sync_copy(src_hbm.at[pl.ds(base, CPT)], srcv)
        pltpu.sync_copy(dst_hbm.at[pl.ds(base, CPT)], dstv)
        _zero_vmem_2d(buf, CHUNK, D)
        rbase = s * SLAB
        for k in range(4):
            pltpu.sync_copy(buf, accs.at[pl.ds(rbase + 128 * k, 128)])
        pltpu.sync_copy(buf.at[pl.ds(0, SLAB - 512)],
                        accs.at[pl.ds(rbase + 512, SLAB - 512)])
        plsc.subcore_barrier()

        def body(i, _):
            pltpu.async_copy(h_hbm.at[srcv.at[i]], buf, sem).wait()
            pltpu.sync_copy(buf, accs.at[dstv.at[i]], add=True)
            return 0

        lax.fori_loop(0, CPT, body, 0)
        plsc.subcore_barrier()
        pltpu.sync_copy(accs.at[pl.ds(rbase, SLAB)],
                        acc_hbm.at[c, pl.ds(rbase, SLAB)])

    return _sc_agg


_sc_agg128 = _make_sc_agg(128)
_sc_agg64 = _make_sc_agg(64)


def _tc_a(x_ref, w_ref, deg_ref, h1p_ref, dis_ref):
    deg = deg_ref[0] + deg_ref[1] + 1.0
    dis = lax.rsqrt(deg)
    dis_ref[...] = dis
    h = jnp.dot(x_ref[...], w_ref[...], preferred_element_type=jnp.float32)
    h1p_ref[...] = h * dis


def _tc_b(acc_ref, h1p_ref, dis_ref, b1_ref, w2_ref, h2p_ref):
    dis = dis_ref[...]
    tot = acc_ref[0] + acc_ref[1] + h1p_ref[...]
    a1 = jnp.maximum(tot * dis + b1_ref[...], 0.0)
    h2p_ref[...] = jnp.dot(a1, w2_ref[...], preferred_element_type=jnp.float32) * dis


def _tc_c(acc_ref, h2p_ref, dis_ref, b2_ref, out_ref):
    tot = acc_ref[0] + acc_ref[1] + h2p_ref[...]
    out_ref[...] = tot * dis_ref[...] + b2_ref[...]


@jax.jit
def kernel(x, edge_index, W1, b1, W2, b2):
    src = edge_index[0].astype(jnp.int32)
    dst = edge_index[1].astype(jnp.int32)
    pad = jnp.full((EPAD - E,), N, dtype=jnp.int32)
    src2d = jnp.concatenate([src, pad]).reshape(NCHUNKS, CHUNK)
    dst2d = jnp.concatenate([dst, pad]).reshape(NCHUNKS, CHUNK)
    x_pad = jnp.pad(x, ((0, NPAD - N), (0, 0)))

    deg2 = _sc_deg(dst2d)
    deg2 = deg2.reshape(NC, NPAD, 1)

    h1p, dis = pl.pallas_call(
        _tc_a,
        out_shape=[
            jax.ShapeDtypeStruct((NPAD, 128), jnp.float32),
            jax.ShapeDtypeStruct((NPAD, 1), jnp.float32),
        ],
    )(x_pad, W1, deg2)

    acc1 = _sc_agg128(h1p, src2d, dst2d)

    h2p = pl.pallas_call(
        _tc_b,
        out_shape=jax.ShapeDtypeStruct((NPAD, 64), jnp.float32),
    )(acc1, h1p, dis, b1.reshape(1, 128), W2)

    acc2 = _sc_agg64(h2p, src2d, dst2d)

    out_pad = pl.pallas_call(
        _tc_c,
        out_shape=jax.ShapeDtypeStruct((NPAD, 64), jnp.float32),
    )(acc2, h2p, dis, b2.reshape(1, 64))

    return out_pad[:N]


# speedup vs baseline: 11.1382x; 11.1382x over previous
import functools

import jax
import jax.numpy as jnp
from jax import lax
from jax.experimental import pallas as pl
from jax.experimental.pallas import tpu as pltpu
from jax.experimental.pallas import tpu_sc as plsc

N = 10000
E = 320000
NC = 2
NS = 16
NW = NC * NS
CHUNK = 128
CPT = 80
EPAD = NW * CPT * CHUNK
NCHUNKS = EPAD // CHUNK
NPAD = 10112
SLAB = NPAD // NS

_mesh = plsc.VectorSubcoreMesh(core_axis_name="c", subcore_axis_name="s")


def _wid_base(cpt):
    c = lax.axis_index("c")
    s = lax.axis_index("s")
    return (c * NS + s) * cpt, c, s


def _zero_vmem_2d(buf, rows, cols):
    z = jnp.zeros((16,), jnp.float32)

    def body(r, _):
        for cc in range(cols // 16):
            buf[r, pl.ds(cc * 16, 16)] = z
        return 0

    lax.fori_loop(0, rows, body, 0)


@functools.partial(
    pl.kernel,
    out_type=[
        jax.ShapeDtypeStruct((NPAD,), jnp.float32),
        jax.ShapeDtypeStruct((NPAD,), jnp.float32),
    ],
    mesh=_mesh,
    scratch_types=[
        pltpu.VMEM((CPT, CHUNK), jnp.int32),
        pltpu.VMEM((CHUNK,), jnp.float32),
        pltpu.VMEM((SLAB,), jnp.float32),
        pltpu.VMEM_SHARED((NPAD,), jnp.float32),
    ],
)
def _sc_deg(dst_hbm, deg0_hbm, deg1_hbm, dstv, ones_v, zslab, dacc):
    base, c, s = _wid_base(CPT)
    pltpu.sync_copy(dst_hbm.at[pl.ds(base, CPT)], dstv)
    one = jnp.ones((16,), jnp.float32)
    z = jnp.zeros((16,), jnp.float32)
    for i in range(CHUNK // 16):
        ones_v[pl.ds(i * 16, 16)] = one

    def zbody(i, _):
        zslab[pl.ds(i * 16, 16)] = z
        return 0

    lax.fori_loop(0, SLAB // 16, zbody, 0)
    pltpu.sync_copy(zslab, dacc.at[pl.ds(s * SLAB, SLAB)])
    plsc.subcore_barrier()

    def body(i, _):
        pltpu.sync_copy(ones_v, dacc.at[dstv.at[i]], add=True)
        return 0

    lax.fori_loop(0, CPT, body, 0)
    plsc.subcore_barrier()

    pltpu.sync_copy(dacc.at[pl.ds(s * SLAB, SLAB)], zslab)

    @pl.when(c == 0)
    def _():
        pltpu.sync_copy(zslab, deg0_hbm.at[pl.ds(s * SLAB, SLAB)])

    @pl.when(c == 1)
    def _():
        pltpu.sync_copy(zslab, deg1_hbm.at[pl.ds(s * SLAB, SLAB)])


def _make_sc_agg(D):
    @functools.partial(
        pl.kernel,
        out_type=jax.ShapeDtypeStruct((NC, NPAD, D), jnp.float32),
        mesh=_mesh,
        scratch_types=[
            pltpu.VMEM((CPT, CHUNK), jnp.int32),
            pltpu.VMEM((CPT, CHUNK), jnp.int32),
            pltpu.VMEM((CHUNK, D), jnp.float32),
            pltpu.VMEM_SHARED((NPAD, D), jnp.float32),
            pltpu.SemaphoreType.DMA,
        ],
        compiler_params=pltpu.CompilerParams(use_tc_tiling_on_sc=False),
    )
    def _sc_agg(h_hbm, src_hbm, dst_hbm, acc_hbm, srcv, dstv, buf, accs, sem):
        base, c, s = _wid_base(CPT)
        pltpu.sync_copy(src_hbm.at[pl.ds(base, CPT)], srcv)
        pltpu.sync_copy(dst_hbm.at[pl.ds(base, CPT)], dstv)
        _zero_vmem_2d(buf, CHUNK, D)
        rbase = s * SLAB
        for k in range(4):
            pltpu.sync_copy(buf, accs.at[pl.ds(rbase + 128 * k, 128)])
        pltpu.sync_copy(buf.at[pl.ds(0, SLAB - 512)],
                        accs.at[pl.ds(rbase + 512, SLAB - 512)])
        plsc.subcore_barrier()

        def body(i, _):
            pltpu.async_copy(h_hbm.at[srcv.at[i]], buf, sem).wait()
            pltpu.sync_copy(buf, accs.at[dstv.at[i]], add=True)
            return 0

        lax.fori_loop(0, CPT, body, 0)
        plsc.subcore_barrier()
        for k in range(5):
            rows = 128 if k < 4 else SLAB - 512
            pltpu.sync_copy(accs.at[pl.ds(rbase + 128 * k, rows)],
                            buf.at[pl.ds(0, rows)])
            pltpu.sync_copy(buf.at[pl.ds(0, rows)],
                            acc_hbm.at[c, pl.ds(rbase + 128 * k, rows)])

    return _sc_agg


_sc_agg128 = _make_sc_agg(128)
_sc_agg64 = _make_sc_agg(64)


def _tc_a(x_ref, w_ref, deg0_ref, deg1_ref, h1p_ref, dis_ref):
    deg = deg0_ref[...] + deg1_ref[...] + 1.0
    dis = lax.rsqrt(deg)
    dis_ref[...] = dis
    h = jnp.dot(x_ref[...], w_ref[...], preferred_element_type=jnp.float32)
    h1p_ref[...] = h * dis


def _tc_b(acc_ref, h1p_ref, dis_ref, b1_ref, w2_ref, h2p_ref):
    dis = dis_ref[...]
    tot = acc_ref[0] + acc_ref[1] + h1p_ref[...]
    a1 = jnp.maximum(tot * dis + b1_ref[...], 0.0)
    h2p_ref[...] = jnp.dot(a1, w2_ref[...], preferred_element_type=jnp.float32) * dis


def _tc_c(acc_ref, h2p_ref, dis_ref, b2_ref, out_ref):
    tot = acc_ref[0] + acc_ref[1] + h2p_ref[...]
    out_ref[...] = tot * dis_ref[...] + b2_ref[...]


@jax.jit
def kernel(x, edge_index, W1, b1, W2, b2):
    src = edge_index[0].astype(jnp.int32)
    dst = edge_index[1].astype(jnp.int32)
    pad = jnp.full((EPAD - E,), N, dtype=jnp.int32)
    src2d = jnp.concatenate([src, pad]).reshape(NCHUNKS, CHUNK)
    dst2d = jnp.concatenate([dst, pad]).reshape(NCHUNKS, CHUNK)
    x_pad = jnp.pad(x, ((0, NPAD - N), (0, 0)))

    deg0, deg1 = _sc_deg(dst2d)

    h1p, dis = pl.pallas_call(
        _tc_a,
        out_shape=[
            jax.ShapeDtypeStruct((NPAD, 128), jnp.float32),
            jax.ShapeDtypeStruct((NPAD, 1), jnp.float32),
        ],
    )(x_pad, W1, deg0.reshape(NPAD, 1), deg1.reshape(NPAD, 1))

    acc1 = _sc_agg128(h1p, src2d, dst2d)

    h2p = pl.pallas_call(
        _tc_b,
        out_shape=jax.ShapeDtypeStruct((NPAD, 64), jnp.float32),
    )(acc1, h1p, dis, b1.reshape(1, 128), W2)

    acc2 = _sc_agg64(h2p, src2d, dst2d)

    out_pad = pl.pallas_call(
        _tc_c,
        out_shape=jax.ShapeDtypeStruct((NPAD, 64), jnp.float32),
    )(acc2, h2p, dis, b2.reshape(1, 64))

    return out_pad[:N]


# speedup vs baseline: 12.0601x; 1.0828x over previous
import functools

import jax
import jax.numpy as jnp
from jax import lax
from jax.experimental import pallas as pl
from jax.experimental.pallas import tpu as pltpu
from jax.experimental.pallas import tpu_sc as plsc

N = 10000
E = 320000
NC = 2
NS = 16
NW = NC * NS
CHUNK = 128
CPT = 80
EPAD = NW * CPT * CHUNK
NCHUNKS = EPAD // CHUNK
NPAD = 10112
SLAB = NPAD // NS

_mesh = plsc.VectorSubcoreMesh(core_axis_name="c", subcore_axis_name="s")


def _wid_base(cpt):
    c = lax.axis_index("c")
    s = lax.axis_index("s")
    return (c * NS + s) * cpt, c, s


def _zero_vmem_2d(buf, rows, cols):
    z = jnp.zeros((16,), jnp.float32)

    def body(r, _):
        for cc in range(cols // 16):
            buf[r, pl.ds(cc * 16, 16)] = z
        return 0

    lax.fori_loop(0, rows, body, 0)


@functools.partial(
    pl.kernel,
    out_type=[
        jax.ShapeDtypeStruct((NPAD,), jnp.float32),
        jax.ShapeDtypeStruct((NPAD,), jnp.float32),
    ],
    mesh=_mesh,
    scratch_types=[
        pltpu.VMEM((CPT, CHUNK), jnp.int32),
        pltpu.VMEM((CHUNK,), jnp.float32),
        pltpu.VMEM((SLAB,), jnp.float32),
        pltpu.VMEM_SHARED((NPAD,), jnp.float32),
    ],
)
def _sc_deg(dst_hbm, deg0_hbm, deg1_hbm, dstv, ones_v, zslab, dacc):
    base, c, s = _wid_base(CPT)
    pltpu.sync_copy(dst_hbm.at[pl.ds(base, CPT)], dstv)
    one = jnp.ones((16,), jnp.float32)
    z = jnp.zeros((16,), jnp.float32)
    for i in range(CHUNK // 16):
        ones_v[pl.ds(i * 16, 16)] = one

    def zbody(i, _):
        zslab[pl.ds(i * 16, 16)] = z
        return 0

    lax.fori_loop(0, SLAB // 16, zbody, 0)
    pltpu.sync_copy(zslab, dacc.at[pl.ds(s * SLAB, SLAB)])
    plsc.subcore_barrier()

    def body(i, _):
        pltpu.sync_copy(ones_v, dacc.at[dstv.at[i]], add=True)
        return 0

    lax.fori_loop(0, CPT, body, 0)
    plsc.subcore_barrier()

    pltpu.sync_copy(dacc.at[pl.ds(s * SLAB, SLAB)], zslab)

    @pl.when(c == 0)
    def _():
        pltpu.sync_copy(zslab, deg0_hbm.at[pl.ds(s * SLAB, SLAB)])

    @pl.when(c == 1)
    def _():
        pltpu.sync_copy(zslab, deg1_hbm.at[pl.ds(s * SLAB, SLAB)])


def _make_sc_agg(D):
    @functools.partial(
        pl.kernel,
        out_type=jax.ShapeDtypeStruct((NC, NPAD, D), jnp.float32),
        mesh=_mesh,
        scratch_types=[
            pltpu.VMEM((CPT, CHUNK), jnp.int32),
            pltpu.VMEM((CHUNK,), jnp.int32),
            pltpu.VMEM((CHUNK,), jnp.int32),
            pltpu.VMEM((CHUNK,), jnp.int32),
            pltpu.VMEM((CHUNK,), jnp.int32),
            pltpu.VMEM((CHUNK, D), jnp.float32),
            pltpu.VMEM((CHUNK, D), jnp.float32),
            pltpu.VMEM_SHARED((NPAD, D), jnp.float32),
            pltpu.SemaphoreType.DMA,
            pltpu.SemaphoreType.DMA,
            pltpu.SemaphoreType.DMA,
            pltpu.SemaphoreType.DMA,
            pltpu.SemaphoreType.DMA,
            pltpu.SemaphoreType.DMA,
            pltpu.SemaphoreType.DMA,
            pltpu.SemaphoreType.DMA,
        ],
        compiler_params=pltpu.CompilerParams(use_tc_tiling_on_sc=False),
    )
    def _sc_agg(h_hbm, src_hbm, dst_hbm, acc_hbm, dstv,
                si0, si1, si2, si3, b0, b1, accs,
                i0, i1, i2, i3, g0, g1, s0, s1):
        si = [si0, si1, si2, si3]
        isem = [i0, i1, i2, i3]
        base, c, s = _wid_base(CPT)
        pltpu.sync_copy(dst_hbm.at[pl.ds(base, CPT)], dstv)
        _zero_vmem_2d(b0, CHUNK, D)
        rbase = s * SLAB
        for k in range(4):
            pltpu.sync_copy(b0, accs.at[pl.ds(rbase + 128 * k, 128)])
        pltpu.sync_copy(b0.at[pl.ds(0, SLAB - 512)],
                        accs.at[pl.ds(rbase + 512, SLAB - 512)])
        plsc.subcore_barrier()

        for j in range(4):
            pltpu.async_copy(src_hbm.at[base + j], si[j], isem[j])
        pltpu.make_async_copy(src_hbm.at[base], si0, i0).wait()
        pltpu.async_copy(h_hbm.at[si0], b0, g0)

        def step(it, _):
            k4 = it * 4
            for j in range(4):
                k = k4 + j
                bp, bq = (b0, b1) if j % 2 == 0 else (b1, b0)
                gp, gq = (g0, g1) if j % 2 == 0 else (g1, g0)
                sp, sq = (s0, s1) if j % 2 == 0 else (s1, s0)
                jn = (j + 1) % 4
                pltpu.make_async_copy(h_hbm.at[si[j]], bp, gp).wait()
                pltpu.async_copy(bp, accs.at[dstv.at[k]], sp, add=True)
                if j == 0:
                    @pl.when(it > 0)
                    def _():
                        pltpu.make_async_copy(bq, accs.at[dstv.at[k]], sq).wait()
                else:
                    pltpu.make_async_copy(bq, accs.at[dstv.at[k]], sq).wait()

                @pl.when(k + 1 < CPT)
                def _():
                    pltpu.make_async_copy(src_hbm.at[base], si[jn], isem[jn]).wait()
                    pltpu.async_copy(h_hbm.at[si[jn]], bq, gq)

                @pl.when(k + 4 < CPT)
                def _():
                    pltpu.async_copy(src_hbm.at[base + k + 4], si[j], isem[j])
            return 0

        lax.fori_loop(0, CPT // 4, step, 0)
        pltpu.make_async_copy(b1, accs.at[dstv.at[0]], s1).wait()
        plsc.subcore_barrier()
        for k in range(5):
            rows = 128 if k < 4 else SLAB - 512
            pltpu.sync_copy(accs.at[pl.ds(rbase + 128 * k, rows)],
                            b0.at[pl.ds(0, rows)])
            pltpu.sync_copy(b0.at[pl.ds(0, rows)],
                            acc_hbm.at[c, pl.ds(rbase + 128 * k, rows)])

    return _sc_agg


_sc_agg128 = _make_sc_agg(128)
_sc_agg64 = _make_sc_agg(64)


def _tc_a(x_ref, w_ref, deg0_ref, deg1_ref, h1p_ref, dis_ref):
    deg = deg0_ref[...] + deg1_ref[...] + 1.0
    dis = lax.rsqrt(deg)
    dis_ref[...] = dis
    h = jnp.dot(x_ref[...], w_ref[...], preferred_element_type=jnp.float32)
    h1p_ref[...] = h * dis


def _tc_b(acc_ref, h1p_ref, dis_ref, b1_ref, w2_ref, h2p_ref):
    dis = dis_ref[...]
    tot = acc_ref[0] + acc_ref[1] + h1p_ref[...]
    a1 = jnp.maximum(tot * dis + b1_ref[...], 0.0)
    h2p_ref[...] = jnp.dot(a1, w2_ref[...], preferred_element_type=jnp.float32) * dis


def _tc_c(acc_ref, h2p_ref, dis_ref, b2_ref, out_ref):
    tot = acc_ref[0] + acc_ref[1] + h2p_ref[...]
    out_ref[...] = tot * dis_ref[...] + b2_ref[...]


@jax.jit
def kernel(x, edge_index, W1, b1, W2, b2):
    src = edge_index[0].astype(jnp.int32)
    dst = edge_index[1].astype(jnp.int32)
    pad = jnp.full((EPAD - E,), N, dtype=jnp.int32)
    src2d = jnp.concatenate([src, pad]).reshape(NCHUNKS, CHUNK)
    dst2d = jnp.concatenate([dst, pad]).reshape(NCHUNKS, CHUNK)
    x_pad = jnp.pad(x, ((0, NPAD - N), (0, 0)))

    deg0, deg1 = _sc_deg(dst2d)

    h1p, dis = pl.pallas_call(
        _tc_a,
        out_shape=[
            jax.ShapeDtypeStruct((NPAD, 128), jnp.float32),
            jax.ShapeDtypeStruct((NPAD, 1), jnp.float32),
        ],
    )(x_pad, W1, deg0.reshape(NPAD, 1), deg1.reshape(NPAD, 1))

    acc1 = _sc_agg128(h1p, src2d, dst2d)

    h2p = pl.pallas_call(
        _tc_b,
        out_shape=jax.ShapeDtypeStruct((NPAD, 64), jnp.float32),
    )(acc1, h1p, dis, b1.reshape(1, 128), W2)

    acc2 = _sc_agg64(h2p, src2d, dst2d)

    out_pad = pl.pallas_call(
        _tc_c,
        out_shape=jax.ShapeDtypeStruct((NPAD, 64), jnp.float32),
    )(acc2, h2p, dis, b2.reshape(1, 64))

    return out_pad[:N]


# speedup vs baseline: 30.6969x; 2.5453x over previous
import functools

import jax
import jax.numpy as jnp
from jax import lax
from jax.experimental import pallas as pl
from jax.experimental.pallas import tpu as pltpu
from jax.experimental.pallas import tpu_sc as plsc

N = 10000
E = 320000
NC = 2
NS = 16
NW = NC * NS
CHUNK = 128
CPT = 80
EPAD = NW * CPT * CHUNK
NCHUNKS = EPAD // CHUNK
NPAD = 10112
SLAB = NPAD // NS

_mesh = plsc.VectorSubcoreMesh(core_axis_name="c", subcore_axis_name="s")


def _wid_base(cpt):
    c = lax.axis_index("c")
    s = lax.axis_index("s")
    return (c * NS + s) * cpt, c, s


def _zero_vmem_2d(buf, rows, cols):
    z = jnp.zeros((16,), jnp.float32)

    def body(r, _):
        for cc in range(cols // 16):
            buf[r, pl.ds(cc * 16, 16)] = z
        return 0

    lax.fori_loop(0, rows, body, 0)


@functools.partial(
    pl.kernel,
    out_type=[
        jax.ShapeDtypeStruct((NPAD,), jnp.float32),
        jax.ShapeDtypeStruct((NPAD,), jnp.float32),
    ],
    mesh=_mesh,
    scratch_types=[
        pltpu.VMEM((CPT, CHUNK), jnp.int32),
        pltpu.VMEM((CHUNK,), jnp.float32),
        pltpu.VMEM((SLAB,), jnp.float32),
        pltpu.VMEM_SHARED((NPAD,), jnp.float32),
    ],
)
def _sc_deg(dst_hbm, deg0_hbm, deg1_hbm, dstv, ones_v, zslab, dacc):
    base, c, s = _wid_base(CPT)
    pltpu.sync_copy(dst_hbm.at[pl.ds(base, CPT)], dstv)
    one = jnp.ones((16,), jnp.float32)
    z = jnp.zeros((16,), jnp.float32)
    for i in range(CHUNK // 16):
        ones_v[pl.ds(i * 16, 16)] = one

    def zbody(i, _):
        zslab[pl.ds(i * 16, 16)] = z
        return 0

    lax.fori_loop(0, SLAB // 16, zbody, 0)
    pltpu.sync_copy(zslab, dacc.at[pl.ds(s * SLAB, SLAB)])
    plsc.subcore_barrier()

    def body(i, _):
        pltpu.sync_copy(ones_v, dacc.at[dstv.at[i]], add=True)
        return 0

    lax.fori_loop(0, CPT, body, 0)
    plsc.subcore_barrier()

    pltpu.sync_copy(dacc.at[pl.ds(s * SLAB, SLAB)], zslab)

    @pl.when(c == 0)
    def _():
        pltpu.sync_copy(zslab, deg0_hbm.at[pl.ds(s * SLAB, SLAB)])

    @pl.when(c == 1)
    def _():
        pltpu.sync_copy(zslab, deg1_hbm.at[pl.ds(s * SLAB, SLAB)])


def _make_sc_agg(D):
    @functools.partial(
        pl.kernel,
        out_type=jax.ShapeDtypeStruct((NC, NPAD, D), jnp.float32),
        mesh=_mesh,
        scratch_types=[
            pltpu.VMEM((CPT, CHUNK), jnp.int32),
            pltpu.VMEM((CHUNK,), jnp.int32),
            pltpu.VMEM((CHUNK,), jnp.int32),
            pltpu.VMEM((CHUNK,), jnp.int32),
            pltpu.VMEM((CHUNK,), jnp.int32),
            pltpu.VMEM((CHUNK, D), jnp.float32),
            pltpu.VMEM((CHUNK, D), jnp.float32),
            pltpu.VMEM_SHARED((NPAD, D), jnp.float32),
            pltpu.SemaphoreType.DMA,
            pltpu.SemaphoreType.DMA,
            pltpu.SemaphoreType.DMA,
            pltpu.SemaphoreType.DMA,
            pltpu.SemaphoreType.DMA,
            pltpu.SemaphoreType.DMA,
            pltpu.SemaphoreType.DMA,
            pltpu.SemaphoreType.DMA,
        ],
        compiler_params=pltpu.CompilerParams(use_tc_tiling_on_sc=False),
    )
    def _sc_agg(h_hbm, src_hbm, dst_hbm, acc_hbm, dstv,
                si0, si1, si2, si3, b0, b1, accs,
                i0, i1, i2, i3, g0, g1, s0, s1):
        si = [si0, si1, si2, si3]
        isem = [i0, i1, i2, i3]
        base, c, s = _wid_base(CPT)
        pltpu.sync_copy(dst_hbm.at[pl.ds(base, CPT)], dstv)
        _zero_vmem_2d(b0, CHUNK, D)
        rbase = s * SLAB
        for k in range(4):
            pltpu.sync_copy(b0, accs.at[pl.ds(rbase + 128 * k, 128)])
        pltpu.sync_copy(b0.at[pl.ds(0, SLAB - 512)],
                        accs.at[pl.ds(rbase + 512, SLAB - 512)])
        plsc.subcore_barrier()

        for j in range(4):
            pltpu.async_copy(src_hbm.at[base + j], si[j], isem[j])
        pltpu.make_async_copy(src_hbm.at[base], si0, i0).wait()
        pltpu.async_copy(h_hbm.at[si0], b0, g0)

        def step(it, _):
            k4 = it * 4
            for j in range(4):
                k = k4 + j
                bp, bq = (b0, b1) if j % 2 == 0 else (b1, b0)
                gp, gq = (g0, g1) if j % 2 == 0 else (g1, g0)
                sp, sq = (s0, s1) if j % 2 == 0 else (s1, s0)
                jn = (j + 1) % 4
                pltpu.make_async_copy(h_hbm.at[si[j]], bp, gp).wait()
                pltpu.async_copy(bp, accs.at[dstv.at[k]], sp, add=True)
                if j == 0:
                    @pl.when(it > 0)
                    def _():
                        pltpu.make_async_copy(bq, accs.at[dstv.at[k]], sq).wait()
                else:
                    pltpu.make_async_copy(bq, accs.at[dstv.at[k]], sq).wait()

                @pl.when(k + 1 < CPT)
                def _():
                    pltpu.make_async_copy(src_hbm.at[base], si[jn], isem[jn]).wait()
                    pltpu.async_copy(h_hbm.at[si[jn]], bq, gq)

                @pl.when(k + 4 < CPT)
                def _():
                    pltpu.async_copy(src_hbm.at[base + k + 4], si[j], isem[j])
            return 0

        lax.fori_loop(0, CPT // 4, step, 0)
        pltpu.make_async_copy(b1, accs.at[dstv.at[0]], s1).wait()
        plsc.subcore_barrier()
        for k in range(5):
            rows = 128 if k < 4 else SLAB - 512
            pltpu.sync_copy(accs.at[pl.ds(rbase + 128 * k, rows)],
                            b0.at[pl.ds(0, rows)])
            pltpu.sync_copy(b0.at[pl.ds(0, rows)],
                            acc_hbm.at[c, pl.ds(rbase + 128 * k, rows)])

    return _sc_agg


_sc_agg128 = _make_sc_agg(128)
_sc_agg64 = _make_sc_agg(64)


def _tc_a(x_ref, w_ref, deg0_ref, deg1_ref, h1p_ref, dis_ref):
    deg = deg0_ref[...] + deg1_ref[...] + 1.0
    dis = lax.rsqrt(deg)
    dis_ref[...] = dis
    h = jnp.dot(x_ref[...], w_ref[...], preferred_element_type=jnp.float32)
    h1p_ref[...] = h * dis


def _tc_b(acc_ref, h1p_ref, dis_ref, b1_ref, w2_ref, h2p_ref):
    dis = dis_ref[...]
    tot = acc_ref[0] + acc_ref[1] + h1p_ref[...]
    a1 = jnp.maximum(tot * dis + b1_ref[...], 0.0)
    h2p_ref[...] = jnp.dot(a1, w2_ref[...], preferred_element_type=jnp.float32) * dis


def _tc_c(acc_ref, h2p_ref, dis_ref, b2_ref, out_ref):
    tot = acc_ref[0] + acc_ref[1] + h2p_ref[...]
    out_ref[...] = tot * dis_ref[...] + b2_ref[...]


@jax.jit
def kernel(x, edge_index, W1, b1, W2, b2):
    src = edge_index[0].astype(jnp.int32)
    dst = edge_index[1].astype(jnp.int32)
    pad = N + jnp.arange(EPAD - E, dtype=jnp.int32) % (NPAD - N)
    src2d = jnp.concatenate([src, pad]).reshape(NCHUNKS, CHUNK)
    dst2d = jnp.concatenate([dst, pad]).reshape(NCHUNKS, CHUNK)
    x_pad = jnp.pad(x, ((0, NPAD - N), (0, 0)))

    deg0, deg1 = _sc_deg(dst2d)

    h1p, dis = pl.pallas_call(
        _tc_a,
        out_shape=[
            jax.ShapeDtypeStruct((NPAD, 128), jnp.float32),
            jax.ShapeDtypeStruct((NPAD, 1), jnp.float32),
        ],
    )(x_pad, W1, deg0.reshape(NPAD, 1), deg1.reshape(NPAD, 1))

    acc1 = _sc_agg128(h1p, src2d, dst2d)

    h2p = pl.pallas_call(
        _tc_b,
        out_shape=jax.ShapeDtypeStruct((NPAD, 64), jnp.float32),
    )(acc1, h1p, dis, b1.reshape(1, 128), W2)

    acc2 = _sc_agg64(h2p, src2d, dst2d)

    out_pad = pl.pallas_call(
        _tc_c,
        out_shape=jax.ShapeDtypeStruct((NPAD, 64), jnp.float32),
    )(acc2, h2p, dis, b2.reshape(1, 64))

    return out_pad[:N]


# speedup vs baseline: 33.6380x; 1.0958x over previous
import functools

import jax
import jax.numpy as jnp
from jax import lax
from jax.experimental import pallas as pl
from jax.experimental.pallas import tpu as pltpu
from jax.experimental.pallas import tpu_sc as plsc

N = 10000
E = 320000
NC = 2
NS = 16
NW = NC * NS
EPT = E // NW
CH = 80
NCH = EPT // CH
NIT = (NCH - 1) // 4
NPAD = 10112
SLAB = NPAD // NS

_mesh = plsc.VectorSubcoreMesh(core_axis_name="c", subcore_axis_name="s")


def _zero_vmem_2d(buf, rows, cols):
    z = jnp.zeros((16,), jnp.float32)

    def body(r, _):
        for cc in range(cols // 16):
            buf[r, pl.ds(cc * 16, 16)] = z
        return 0

    lax.fori_loop(0, rows, body, 0)


def _zero_slab(buf, accs, rbase, rows_total, rows_buf):
    nfull, rem = divmod(rows_total, rows_buf)
    for k in range(nfull):
        pltpu.sync_copy(buf, accs.at[pl.ds(rbase + rows_buf * k, rows_buf)])
    if rem:
        pltpu.sync_copy(buf.at[pl.ds(0, rem)],
                        accs.at[pl.ds(rbase + rows_buf * nfull, rem)])


@functools.partial(
    pl.kernel,
    out_type=[
        jax.ShapeDtypeStruct((NPAD,), jnp.float32),
        jax.ShapeDtypeStruct((NPAD,), jnp.float32),
    ],
    mesh=_mesh,
    scratch_types=[
        pltpu.VMEM((CH,), jnp.int32),
        pltpu.VMEM((CH,), jnp.int32),
        pltpu.VMEM((CH,), jnp.int32),
        pltpu.VMEM((CH,), jnp.int32),
        pltpu.VMEM((CH,), jnp.float32),
        pltpu.VMEM((SLAB,), jnp.float32),
        pltpu.VMEM_SHARED((NPAD,), jnp.float32),
        pltpu.SemaphoreType.DMA,
        pltpu.SemaphoreType.DMA,
        pltpu.SemaphoreType.DMA,
        pltpu.SemaphoreType.DMA,
        pltpu.SemaphoreType.DMA,
        pltpu.SemaphoreType.DMA,
        pltpu.SemaphoreType.DMA,
        pltpu.SemaphoreType.DMA,
    ],
    compiler_params=pltpu.CompilerParams(use_tc_tiling_on_sc=False),
)
def _sc_deg(ei_hbm, deg0_hbm, deg1_hbm,
            d0, d1, d2, d3, ones_v, zslab, dacc,
            sd0, sd1, sd2, sd3, ss0, ss1, ss2, ss3):
    di = [d0, d1, d2, d3]
    ds = [sd0, sd1, sd2, sd3]
    ss = [ss0, ss1, ss2, ss3]
    c = lax.axis_index("c")
    s = lax.axis_index("s")
    ebase = (c * NS + s) * EPT
    one = jnp.ones((16,), jnp.float32)
    z = jnp.zeros((16,), jnp.float32)
    for i in range(CH // 16):
        ones_v[pl.ds(i * 16, 16)] = one

    def zbody(i, _):
        zslab[pl.ds(i * 16, 16)] = z
        return 0

    lax.fori_loop(0, SLAB // 16, zbody, 0)
    pltpu.sync_copy(zslab, dacc.at[pl.ds(s * SLAB, SLAB)])
    plsc.subcore_barrier()

    def _stage(j, k):
        pltpu.async_copy(ei_hbm.at[pl.ds(E + ebase + k * CH, CH)], di[j], ds[j])

    def _wait_stage(j):
        pltpu.make_async_copy(ei_hbm.at[pl.ds(E + ebase, CH)], di[j], ds[j]).wait()

    for j in range(4):
        _stage(j, j)

    def step(it, _):
        k0 = it * 4
        for j in range(4):
            k = k0 + j
            _wait_stage(j)
            pltpu.sync_copy(ones_v, dacc.at[di[j]], add=True)

            @pl.when(k + 4 <= NCH - 1)
            def _():
                _stage(j, k + 4)
        return 0

    lax.fori_loop(0, NIT, step, 0)
    _wait_stage(0)
    pltpu.sync_copy(ones_v, dacc.at[di[0]], add=True)
    plsc.subcore_barrier()
    pltpu.sync_copy(dacc.at[pl.ds(s * SLAB, SLAB)], zslab)

    @pl.when(c == 0)
    def _():
        pltpu.sync_copy(zslab, deg0_hbm.at[pl.ds(s * SLAB, SLAB)])

    @pl.when(c == 1)
    def _():
        pltpu.sync_copy(zslab, deg1_hbm.at[pl.ds(s * SLAB, SLAB)])


def _make_sc_agg(D):
    @functools.partial(
        pl.kernel,
        out_type=jax.ShapeDtypeStruct((NC, NPAD, D), jnp.float32),
        mesh=_mesh,
        scratch_types=[
            pltpu.VMEM((CH,), jnp.int32),
            pltpu.VMEM((CH,), jnp.int32),
            pltpu.VMEM((CH,), jnp.int32),
            pltpu.VMEM((CH,), jnp.int32),
            pltpu.VMEM((CH,), jnp.int32),
            pltpu.VMEM((CH,), jnp.int32),
            pltpu.VMEM((CH,), jnp.int32),
            pltpu.VMEM((CH,), jnp.int32),
            pltpu.VMEM((CH, D), jnp.float32),
            pltpu.VMEM((CH, D), jnp.float32),
            pltpu.VMEM((CH, D), jnp.float32),
            pltpu.VMEM((CH, D), jnp.float32),
            pltpu.VMEM_SHARED((NPAD, D), jnp.float32),
            pltpu.SemaphoreType.DMA,
            pltpu.SemaphoreType.DMA,
            pltpu.SemaphoreType.DMA,
            pltpu.SemaphoreType.DMA,
            pltpu.SemaphoreType.DMA,
            pltpu.SemaphoreType.DMA,
            pltpu.SemaphoreType.DMA,
            pltpu.SemaphoreType.DMA,
            pltpu.SemaphoreType.DMA,
            pltpu.SemaphoreType.DMA,
            pltpu.SemaphoreType.DMA,
            pltpu.SemaphoreType.DMA,
            pltpu.SemaphoreType.DMA,
            pltpu.SemaphoreType.DMA,
            pltpu.SemaphoreType.DMA,
            pltpu.SemaphoreType.DMA,
        ],
        compiler_params=pltpu.CompilerParams(use_tc_tiling_on_sc=False),
    )
    def _sc_agg(h_hbm, ei_hbm, acc_hbm,
                s0, s1, s2, s3, d0, d1, d2, d3, b0, b1, b2, b3, accs,
                es0, es1, es2, es3, ds0, ds1, ds2, ds3,
                gs0, gs1, gs2, gs3, ss0, ss1, ss2, ss3):
        si = [s0, s1, s2, s3]
        di = [d0, d1, d2, d3]
        bufs = [b0, b1, b2, b3]
        es = [es0, es1, es2, es3]
        ds = [ds0, ds1, ds2, ds3]
        gs = [gs0, gs1, gs2, gs3]
        ss = [ss0, ss1, ss2, ss3]
        c = lax.axis_index("c")
        s = lax.axis_index("s")
        ebase = (c * NS + s) * EPT
        _zero_vmem_2d(b0, CH, D)
        rbase = s * SLAB
        _zero_slab(b0, accs, rbase, SLAB, CH)
        plsc.subcore_barrier()

        def _stage_src(j, k):
            pltpu.async_copy(ei_hbm.at[pl.ds(ebase + k * CH, CH)], si[j], es[j])

        def _wait_src(j):
            pltpu.make_async_copy(ei_hbm.at[pl.ds(ebase, CH)], si[j], es[j]).wait()

        def _stage_dst(j, k):
            pltpu.async_copy(ei_hbm.at[pl.ds(E + ebase + k * CH, CH)], di[j], ds[j])

        def _wait_dst(j):
            pltpu.make_async_copy(ei_hbm.at[pl.ds(E + ebase, CH)], di[j], ds[j]).wait()

        for j in range(4):
            _stage_src(j, j)
        _stage_dst(0, 0)
        _stage_dst(1, 1)
        _wait_src(0)
        pltpu.async_copy(h_hbm.at[si[0]], b0, gs[0])
        _wait_src(1)
        pltpu.async_copy(h_hbm.at[si[1]], b1, gs[1])

        def step(it, _):
            k0 = it * 4
            for j in range(4):
                k = k0 + j
                j2 = (j + 2) % 4
                pltpu.make_async_copy(h_hbm.at[si[j]], bufs[j], gs[j]).wait()

                @pl.when(k + 4 <= NCH - 1)
                def _():
                    _stage_src(j, k + 4)

                _wait_dst(j)
                pltpu.sync_copy(bufs[j], accs.at[di[j]], add=True)

                @pl.when(k + 2 <= NCH - 1)
                def _():
                    _stage_dst(j2, k + 2)
                    _wait_src(j2)
                    pltpu.async_copy(h_hbm.at[si[j2]], bufs[j2], gs[j2])
            return 0

        lax.fori_loop(0, NIT, step, 0)
        pltpu.make_async_copy(h_hbm.at[si[0]], b0, gs[0]).wait()
        _wait_dst(0)
        pltpu.sync_copy(b0, accs.at[di[0]], add=True)
        plsc.subcore_barrier()
        nfull, rem = divmod(SLAB, CH)
        for k in range(nfull + 1):
            rows = CH if k < nfull else rem
            pltpu.sync_copy(accs.at[pl.ds(rbase + CH * k, rows)],
                            b0.at[pl.ds(0, rows)])
            pltpu.sync_copy(b0.at[pl.ds(0, rows)],
                            acc_hbm.at[c, pl.ds(rbase + CH * k, rows)])

    return _sc_agg


_sc_agg128 = _make_sc_agg(128)
_sc_agg64 = _make_sc_agg(64)


def _tc_a(x_ref, w_ref, deg0_ref, deg1_ref, h1p_ref, dis_ref):
    deg = deg0_ref[...] + deg1_ref[...] + 1.0
    dis = lax.rsqrt(deg)
    dis_ref[...] = dis
    h = jnp.dot(x_ref[...], w_ref[...], preferred_element_type=jnp.float32)
    h1p_ref[pl.ds(0, N)] = h * dis[:N]
    h1p_ref[pl.ds(N, NPAD - N)] = jnp.zeros((NPAD - N, 128), jnp.float32)


def _tc_b(acc_ref, h1p_ref, dis_ref, b1_ref, w2_ref, h2p_ref):
    dis = dis_ref[...]
    tot = acc_ref[0] + acc_ref[1] + h1p_ref[...]
    a1 = jnp.maximum(tot * dis + b1_ref[...], 0.0)
    h2p_ref[...] = jnp.dot(a1, w2_ref[...], preferred_element_type=jnp.float32) * dis


def _tc_c(acc_ref, h2p_ref, dis_ref, b2_ref, out_ref):
    tot = acc_ref[0, pl.ds(0, N)] + acc_ref[1, pl.ds(0, N)] + h2p_ref[pl.ds(0, N)]
    out_ref[...] = tot * dis_ref[pl.ds(0, N)] + b2_ref[...]


@jax.jit
def kernel(x, edge_index, W1, b1, W2, b2):
    ei = edge_index.astype(jnp.int32).reshape(2 * E)

    deg0, deg1 = _sc_deg(ei)

    h1p, dis = pl.pallas_call(
        _tc_a,
        out_shape=[
            jax.ShapeDtypeStruct((NPAD, 128), jnp.float32),
            jax.ShapeDtypeStruct((NPAD, 1), jnp.float32),
        ],
    )(x, W1, deg0.reshape(NPAD, 1), deg1.reshape(NPAD, 1))

    acc1 = _sc_agg128(h1p, ei)

    h2p = pl.pallas_call(
        _tc_b,
        out_shape=jax.ShapeDtypeStruct((NPAD, 64), jnp.float32),
    )(acc1, h1p, dis, b1.reshape(1, 128), W2)

    acc2 = _sc_agg64(h2p, ei)

    out = pl.pallas_call(
        _tc_c,
        out_shape=jax.ShapeDtypeStruct((N, 64), jnp.float32),
    )(acc2, h2p, dis, b2.reshape(1, 64))

    return out
